# Initial kernel scaffold; baseline (speedup 1.0000x reference)
import jax
import jax.numpy as jnp
from jax.experimental import pallas as pl


def kernel(inputs, parameter):
    raise NotImplementedError("write your pallas kernel here")


# speedup vs baseline: 2.8699x; 2.8699x over previous
import functools

import jax
import jax.numpy as jnp
from jax import lax
from jax.experimental import pallas as pl
from jax.experimental.pallas import tpu as pltpu
from jax.experimental.pallas import tpu_sc as plsc


_RBLK = 256
_CBLK = 4096


def _argmax_body(p_ref, out_ref, vmax_ref, vidx_ref):
    j = pl.program_id(1)
    x = p_ref[...]
    bm = jnp.max(x, axis=1, keepdims=True)
    col = jax.lax.broadcasted_iota(jnp.int32, x.shape, 1) + j * _CBLK
    big = jnp.int32(2**31 - 1)
    bi = jnp.min(jnp.where(x == bm, col, big), axis=1, keepdims=True)

    @pl.when(j == 0)
    def _init():
        vmax_ref[...] = bm
        vidx_ref[...] = bi

    @pl.when(j > 0)
    def _acc():
        upd = bm > vmax_ref[...]
        vmax_ref[...] = jnp.where(upd, bm, vmax_ref[...])
        vidx_ref[...] = jnp.where(upd, bi, vidx_ref[...])

    @pl.when(j == pl.num_programs(1) - 1)
    def _fin():
        out_ref[...] = vidx_ref[...]


def _rowwise_argmax(parameter):
    n_rows, n_cols = parameter.shape
    grid = (n_rows // _RBLK, n_cols // _CBLK)
    ind2d = pl.pallas_call(
        _argmax_body,
        grid=grid,
        in_specs=[pl.BlockSpec((_RBLK, _CBLK), lambda i, j: (i, j))],
        out_specs=pl.BlockSpec((_RBLK, 1), lambda i, j: (i, 0)),
        out_shape=jax.ShapeDtypeStruct((n_rows, 1), jnp.int32),
        scratch_shapes=[
            pltpu.VMEM((_RBLK, 1), jnp.float32),
            pltpu.VMEM((_RBLK, 1), jnp.int32),
        ],
    )(parameter)
    return ind2d.reshape(n_rows)


def _make_sc_gather(V, D, B):
    info = plsc.get_sparse_core_info()
    NC, NS = info.num_cores, info.num_subcores
    NW = NC * NS
    assert B % (8 * NW) == 0
    b_per_w = B // NW
    mesh = plsc.VectorSubcoreMesh(core_axis_name="c", subcore_axis_name="s")

    @functools.partial(
        pl.kernel,
        mesh=mesh,
        out_type=jax.ShapeDtypeStruct((B, D), jnp.float32),
        scratch_types=[
            pltpu.VMEM((b_per_w,), jnp.int32),
            pltpu.VMEM((b_per_w, D), jnp.float32),
            pltpu.SemaphoreType.DMA,
        ],
    )
    def gather_k(table_hbm, idx_hbm, out_hbm, idx_v, rows_v, sem):
        wid = lax.axis_index("s") * NC + lax.axis_index("c")
        base = wid * b_per_w
        pltpu.sync_copy(idx_hbm.at[pl.ds(base, b_per_w)], idx_v)
        pltpu.async_copy(table_hbm.at[idx_v], rows_v, sem).wait()
        pltpu.sync_copy(rows_v, out_hbm.at[pl.ds(base, b_per_w)])

    return gather_k


def kernel(inputs, parameter):
    ind = _rowwise_argmax(parameter)
    table = inputs.T
    V, D = table.shape
    B = ind.shape[0]
    rows = _make_sc_gather(V, D, B)(table, ind)
    return rows.T


# speedup vs baseline: 3.3956x; 1.1832x over previous
import functools

import jax
import jax.numpy as jnp
from jax import lax
from jax.experimental import pallas as pl
from jax.experimental.pallas import tpu as pltpu
from jax.experimental.pallas import tpu_sc as plsc


_RBLK = 256
_CBLK = 8192


def _argmax_body(p_ref, out_ref, vmax_ref, vidx_ref):
    j = pl.program_id(1)
    x = p_ref[...]
    bm = jnp.max(x, axis=1, keepdims=True)
    col = jax.lax.broadcasted_iota(jnp.int32, x.shape, 1) + j * _CBLK
    big = jnp.int32(2**31 - 1)
    bi = jnp.min(jnp.where(x == bm, col, big), axis=1, keepdims=True)

    @pl.when(j == 0)
    def _init():
        vmax_ref[...] = bm
        vidx_ref[...] = bi

    @pl.when(j > 0)
    def _acc():
        upd = bm > vmax_ref[...]
        vmax_ref[...] = jnp.where(upd, bm, vmax_ref[...])
        vidx_ref[...] = jnp.where(upd, bi, vidx_ref[...])

    @pl.when(j == pl.num_programs(1) - 1)
    def _fin():
        out_ref[...] = vidx_ref[...]


def _rowwise_argmax(parameter):
    n_rows, n_cols = parameter.shape
    grid = (n_rows // _RBLK, n_cols // _CBLK)
    ind2d = pl.pallas_call(
        _argmax_body,
        grid=grid,
        in_specs=[pl.BlockSpec((_RBLK, _CBLK), lambda i, j: (i, j))],
        out_specs=pl.BlockSpec((_RBLK, 1), lambda i, j: (i, 0)),
        out_shape=jax.ShapeDtypeStruct((n_rows, 1), jnp.int32),
        scratch_shapes=[
            pltpu.VMEM((_RBLK, 1), jnp.float32),
            pltpu.VMEM((_RBLK, 1), jnp.int32),
        ],
    )(parameter)
    return ind2d.reshape(n_rows)


def _make_sc_gather(V, D, B):
    info = plsc.get_sparse_core_info()
    NC, NS = info.num_cores, info.num_subcores
    NW = NC * NS
    assert B % (8 * NW) == 0
    b_per_w = B // NW
    mesh = plsc.VectorSubcoreMesh(core_axis_name="c", subcore_axis_name="s")

    @functools.partial(
        pl.kernel,
        mesh=mesh,
        out_type=jax.ShapeDtypeStruct((B, D), jnp.float32),
        scratch_types=[
            pltpu.VMEM((b_per_w,), jnp.int32),
            pltpu.VMEM((b_per_w, D), jnp.float32),
            pltpu.SemaphoreType.DMA,
        ],
    )
    def gather_k(table_hbm, idx_hbm, out_hbm, idx_v, rows_v, sem):
        wid = lax.axis_index("s") * NC + lax.axis_index("c")
        base = wid * b_per_w
        pltpu.sync_copy(idx_hbm.at[pl.ds(base, b_per_w)], idx_v)
        pltpu.async_copy(table_hbm.at[idx_v], rows_v, sem).wait()
        pltpu.sync_copy(rows_v, out_hbm.at[pl.ds(base, b_per_w)])

    return gather_k


def kernel(inputs, parameter):
    ind = _rowwise_argmax(parameter)
    table = inputs.T
    V, D = table.shape
    B = ind.shape[0]
    rows = _make_sc_gather(V, D, B)(table, ind)
    return rows.T


# speedup vs baseline: 3.5792x; 1.0541x over previous
import functools

import jax
import jax.numpy as jnp
from jax import lax
from jax.experimental import pallas as pl
from jax.experimental.pallas import tpu as pltpu
from jax.experimental.pallas import tpu_sc as plsc


_RBLK = 256
_CBLK = 16384


def _argmax_body(p_ref, out_ref, vmax_ref, vidx_ref):
    j = pl.program_id(1)
    x = p_ref[...]
    bm = jnp.max(x, axis=1, keepdims=True)
    col = jax.lax.broadcasted_iota(jnp.int32, x.shape, 1) + j * _CBLK
    big = jnp.int32(2**31 - 1)
    bi = jnp.min(jnp.where(x == bm, col, big), axis=1, keepdims=True)

    @pl.when(j == 0)
    def _init():
        vmax_ref[...] = bm
        vidx_ref[...] = bi

    @pl.when(j > 0)
    def _acc():
        upd = bm > vmax_ref[...]
        vmax_ref[...] = jnp.where(upd, bm, vmax_ref[...])
        vidx_ref[...] = jnp.where(upd, bi, vidx_ref[...])

    @pl.when(j == pl.num_programs(1) - 1)
    def _fin():
        out_ref[...] = vidx_ref[...]


def _rowwise_argmax(parameter):
    n_rows, n_cols = parameter.shape
    grid = (n_rows // _RBLK, n_cols // _CBLK)
    ind2d = pl.pallas_call(
        _argmax_body,
        grid=grid,
        in_specs=[pl.BlockSpec((_RBLK, _CBLK), lambda i, j: (i, j))],
        out_specs=pl.BlockSpec((_RBLK, 1), lambda i, j: (i, 0)),
        out_shape=jax.ShapeDtypeStruct((n_rows, 1), jnp.int32),
        scratch_shapes=[
            pltpu.VMEM((_RBLK, 1), jnp.float32),
            pltpu.VMEM((_RBLK, 1), jnp.int32),
        ],
    )(parameter)
    return ind2d.reshape(n_rows)


def _make_sc_gather(V, D, B):
    info = plsc.get_sparse_core_info()
    NC, NS = info.num_cores, info.num_subcores
    NW = NC * NS
    assert B % (8 * NW) == 0
    b_per_w = B // NW
    mesh = plsc.VectorSubcoreMesh(core_axis_name="c", subcore_axis_name="s")

    @functools.partial(
        pl.kernel,
        mesh=mesh,
        out_type=jax.ShapeDtypeStruct((B, D), jnp.float32),
        scratch_types=[
            pltpu.VMEM((b_per_w,), jnp.int32),
            pltpu.VMEM((b_per_w, D), jnp.float32),
            pltpu.SemaphoreType.DMA,
        ],
    )
    def gather_k(table_hbm, idx_hbm, out_hbm, idx_v, rows_v, sem):
        wid = lax.axis_index("s") * NC + lax.axis_index("c")
        base = wid * b_per_w
        pltpu.sync_copy(idx_hbm.at[pl.ds(base, b_per_w)], idx_v)
        pltpu.async_copy(table_hbm.at[idx_v], rows_v, sem).wait()
        pltpu.sync_copy(rows_v, out_hbm.at[pl.ds(base, b_per_w)])

    return gather_k


def kernel(inputs, parameter):
    ind = _rowwise_argmax(parameter)
    table = inputs.T
    V, D = table.shape
    B = ind.shape[0]
    rows = _make_sc_gather(V, D, B)(table, ind)
    return rows.T
